# baseline (speedup 1.0000x reference)
import functools

import jax
import jax.numpy as jnp
from jax import lax
from jax.experimental import pallas as pl
from jax.experimental.pallas import tpu as pltpu
from jax.experimental.pallas import tpu_sc as plsc

N = 10000
NPAD = 10112
D_IN = 128
H1, C1 = 8, 16
H2, C2 = 1, 2
NEG = -1e30
CHUNK = 128
NTILES = 32
RPT = NPAD // 16

_HIGH = lax.Precision.HIGHEST


def _cdiv(a, b):
    return (a + b - 1) // b


def _prep_body(x_ref, w_ref, asrc_ref, adst_ref, pk_ref, ad_ref, mh_ref):
    x = x_ref[...]
    h = jnp.dot(x, w_ref[...], precision=_HIGH)
    als = jnp.dot(h, asrc_ref[...], precision=_HIGH)
    ald = jnp.dot(h, adst_ref[...], precision=_HIGH)
    pad = jnp.full((NPAD, 16 - H1), NEG, jnp.float32)
    pk_ref[...] = jnp.concatenate([als, pad, h], axis=1)
    ad_ref[...] = jnp.concatenate([ald, pad], axis=1)
    mh = jnp.max(als, axis=0) + jnp.max(ald, axis=0)
    mh_ref[...] = jnp.concatenate([mh, jnp.zeros((16 - H1,), jnp.float32)])[None, :]


def _tc_prep(xp, w, asrc, adst):
    return pl.pallas_call(
        _prep_body,
        out_shape=[
            jax.ShapeDtypeStruct((NPAD, 16 + H1 * C1), jnp.float32),
            jax.ShapeDtypeStruct((NPAD, 16), jnp.float32),
            jax.ShapeDtypeStruct((1, 16), jnp.float32),
        ],
    )(xp, w, asrc, adst)


BR = NPAD // 8


def _mid_body(q_ref, b1_ref, w2_ref, asrc_ref, adst_ref,
              pk_ref, ad_ref, mh_ref, ms_ref, md_ref):
    @pl.when(pl.program_id(0) == 0)
    def _init():
        ms_ref[...] = jnp.full((1, 16), NEG, jnp.float32)
        md_ref[...] = jnp.full((1, 16), NEG, jnp.float32)

    den = q_ref[0, :, :16] + q_ref[1, :, :16]
    feat = q_ref[0, :, 16:] + q_ref[1, :, 16:]
    r = 1.0 / (den[:, :H1] + 1e-16)
    rrep = jnp.reshape(
        jnp.broadcast_to(r[:, :, None], (BR, H1, C1)), (BR, H1 * C1))
    o = feat * rrep + b1_ref[...]
    o = jnp.where(o > 0, o, jnp.exp(jnp.minimum(o, 0.0)) - 1.0)
    h2 = jnp.dot(o, w2_ref[...], precision=_HIGH)
    als = jnp.dot(h2, asrc_ref[...], precision=_HIGH)
    ald = jnp.dot(h2, adst_ref[...], precision=_HIGH)
    negpad = jnp.full((BR, 15), NEG, jnp.float32)
    h2pad = jnp.concatenate(
        [h2, jnp.zeros((BR, 16 - C2), jnp.float32)], axis=1)
    pk_ref[...] = jnp.concatenate([als, negpad, h2pad], axis=1)
    ad_ref[...] = jnp.concatenate([ald, negpad], axis=1)
    ms_ref[...] = jnp.maximum(ms_ref[...], jnp.max(als))
    md_ref[...] = jnp.maximum(md_ref[...], jnp.max(ald))
    mh_ref[...] = ms_ref[...] + md_ref[...]


def _tc_mid(q, b1, w2, asrc2, adst2):
    return pl.pallas_call(
        _mid_body,
        grid=(NPAD // BR,),
        in_specs=[
            pl.BlockSpec((2, BR, 16 + H1 * C1), lambda i: (0, i, 0)),
            pl.BlockSpec((1, H1 * C1), lambda i: (0, 0)),
            pl.BlockSpec((H1 * C1, C2), lambda i: (0, 0)),
            pl.BlockSpec((C2, H2), lambda i: (0, 0)),
            pl.BlockSpec((C2, H2), lambda i: (0, 0)),
        ],
        out_specs=[
            pl.BlockSpec((BR, 32), lambda i: (i, 0)),
            pl.BlockSpec((BR, 16), lambda i: (i, 0)),
            pl.BlockSpec((1, 16), lambda i: (0, 0)),
        ],
        out_shape=[
            jax.ShapeDtypeStruct((NPAD, 32), jnp.float32),
            jax.ShapeDtypeStruct((NPAD, 16), jnp.float32),
            jax.ShapeDtypeStruct((1, 16), jnp.float32),
        ],
        scratch_shapes=[
            pltpu.VMEM((1, 16), jnp.float32),
            pltpu.VMEM((1, 16), jnp.float32),
        ],
    )(q, b1, w2, asrc2, adst2)


def _final_body(q_ref, b2_ref, o_ref):
    den = q_ref[0, :N, 0:1] + q_ref[1, :N, 0:1]
    feat = q_ref[0, :N, 16:16 + C2] + q_ref[1, :N, 16:16 + C2]
    o_ref[...] = feat / (den + 1e-16) + b2_ref[...]


def _tc_final(q, b2):
    return pl.pallas_call(
        _final_body,
        out_shape=jax.ShapeDtypeStruct((N, C2), jnp.float32),
    )(q, b2.reshape(1, C2))


def _sc_fused(epad, npt, hv, chunk, unr):
    nch = npt // chunk
    aw = 16 * (hv + 1)
    mesh = plsc.VectorSubcoreMesh(core_axis_name="c", subcore_axis_name="s")

    @functools.partial(
        pl.kernel,
        out_type=jax.ShapeDtypeStruct((2, NPAD, aw), jnp.float32),
        mesh=mesh,
        compiler_params=pltpu.CompilerParams(use_tc_tiling_on_sc=False),
        scratch_types=[
            pltpu.VMEM((chunk,), jnp.int32),
            pltpu.VMEM((chunk,), jnp.int32),
            pltpu.VMEM((chunk,), jnp.int32),
            pltpu.VMEM((chunk,), jnp.int32),
            pltpu.VMEM((chunk, aw), jnp.float32),
            pltpu.VMEM((chunk, aw), jnp.float32),
            pltpu.VMEM((chunk, 16), jnp.float32),
            pltpu.VMEM((chunk, 16), jnp.float32),
            pltpu.VMEM((1, 16), jnp.float32),
            pltpu.VMEM_SHARED((NPAD, aw), jnp.float32),
            pltpu.SemaphoreType.DMA,
            pltpu.SemaphoreType.DMA,
        ],
    )
    def kern(pk_hbm, ad_hbm, mh_hbm, src_hbm, dst_hbm, z_hbm,
             out_hbm,
             srcv0, srcv1, dstv0, dstv1, pkr0, pkr1, adr0, adr1,
             mhv, acc, sem0, sem1):
        cid = lax.axis_index("c")
        sid = lax.axis_index("s")
        wid = cid * 16 + sid
        pltpu.sync_copy(z_hbm.at[pl.ds(sid * RPT, RPT)],
                        acc.at[pl.ds(sid * RPT, RPT)])
        pltpu.sync_copy(mh_hbm, mhv)
        plsc.subcore_barrier()
        mh = mhv[0, :]

        bufs = [(srcv0, dstv0, pkr0, adr0, sem0),
                (srcv1, dstv1, pkr1, adr1, sem1)]

        def fire(ci, s):
            srcv, dstv, pkr, adr, sem = bufs[s]
            base = wid * npt + ci * chunk
            pltpu.sync_copy(src_hbm.at[pl.ds(base, chunk)], srcv)
            pltpu.sync_copy(dst_hbm.at[pl.ds(base, chunk)], dstv)
            pltpu.async_copy(pk_hbm.at[srcv], pkr, sem)
            pltpu.async_copy(ad_hbm.at[dstv], adr, sem)

        def drain(s):
            srcv, dstv, pkr, adr, sem = bufs[s]
            pltpu.make_async_copy(pk_hbm.at[srcv], pkr, sem).wait()
            pltpu.make_async_copy(ad_hbm.at[dstv], adr, sem).wait()

        def compute(s):
            srcv, dstv, pkr, adr, sem = bufs[s]

            @plsc.parallel_loop(0, chunk, unroll=unr)
            def _edge(e):
                t = pkr[e, pl.ds(0, 16)] + adr[e, :]
                u = jnp.maximum(t, t * 0.2) - mh
                ex = jnp.exp(u)
                pkr[e, pl.ds(0, 16)] = ex
                for v in range(hv):
                    sp = jnp.broadcast_to(ex[v], (16,))
                    pkr[e, pl.ds(16 * (v + 1), 16)] = (
                        pkr[e, pl.ds(16 * (v + 1), 16)] * sp)

            pltpu.sync_copy(pkr, acc.at[dstv], add=True)

        half = nch // 2
        fire(0, 0)

        @pl.loop(0, half)
        def _grp(g):
            fire(2 * g + 1, 1)
            drain(0)
            compute(0)
            fire(jnp.minimum(2 * g + 2, nch - 1), 0)
            drain(1)
            compute(1)

        drain(0)
        plsc.subcore_barrier()
        pltpu.sync_copy(acc.at[pl.ds(sid * RPT, RPT)],
                        out_hbm.at[cid, pl.ds(sid * RPT, RPT)])

    return kern


def _block_attn(a):
    h = a.shape[0]
    eye = jnp.eye(h, dtype=a.dtype)
    return (a[:, :, None] * eye[:, None, :]).reshape(h * a.shape[1], h)


def kernel(x, edge_index, W1, a_src1, a_dst1, b1, W2, a_src2, a_dst2, b2):
    e = edge_index.shape[1]
    etot = e + N
    npt = _cdiv(etot, NTILES * 3584) * 3584
    epad = npt * NTILES

    xp = jnp.concatenate([x, jnp.zeros((NPAD - N, D_IN), jnp.float32)], axis=0)
    loops = jnp.arange(N, dtype=jnp.int32)
    padv = N + jnp.arange(epad - etot, dtype=jnp.int32) % (NPAD - N)
    src = jnp.concatenate([edge_index[0], loops, padv])
    dst = jnp.concatenate([edge_index[1], loops, padv])

    asrc1 = _block_attn(a_src1)
    adst1 = _block_attn(a_dst1)
    asrc2 = _block_attn(a_src2)
    adst2 = _block_attn(a_dst2)

    z144 = jnp.zeros((NPAD, 16 + H1 * C1), jnp.float32)
    z32 = jnp.zeros((NPAD, 32), jnp.float32)

    fused1 = _sc_fused(epad, npt, H1, 112, 8)
    fused2 = _sc_fused(epad, npt, 1, 256, 16)

    pk1, ad1, mh1 = _tc_prep(xp, W1, asrc1, adst1)
    q1 = fused1(pk1, ad1, mh1, src, dst, z144)

    pk2, ad2, mh2 = _tc_mid(q1, b1.reshape(1, H1 * C1), W2, asrc2, adst2)
    q2 = fused2(pk2, ad2, mh2, src, dst, z32)

    return _tc_final(q2, b2)


# speedup vs baseline: 1.2790x; 1.2790x over previous
import functools

import jax
import jax.numpy as jnp
from jax import lax
from jax.experimental import pallas as pl
from jax.experimental.pallas import tpu as pltpu
from jax.experimental.pallas import tpu_sc as plsc

N = 10000
NPAD = 10112
D_IN = 128
H1, C1 = 8, 16
H2, C2 = 1, 2
NEG = -1e30
CHUNK = 128
NTILES = 32
RPT = NPAD // 16

_HIGH = lax.Precision.HIGHEST


def _cdiv(a, b):
    return (a + b - 1) // b


def _prep_body(x_ref, w_ref, asrc_ref, adst_ref, pk_ref, ad_ref, mh_ref):
    x = x_ref[...]
    h = jnp.dot(x, w_ref[...], precision=_HIGH)
    als = jnp.dot(h, asrc_ref[...], precision=_HIGH)
    ald = jnp.dot(h, adst_ref[...], precision=_HIGH)
    pad = jnp.full((NPAD, 16 - H1), NEG, jnp.float32)
    pk_ref[...] = jnp.concatenate([als, pad, h], axis=1)
    ad_ref[...] = jnp.concatenate([ald, pad], axis=1)
    mh = jnp.max(als, axis=0) + jnp.max(ald, axis=0)
    mh_ref[...] = jnp.concatenate([mh, jnp.zeros((16 - H1,), jnp.float32)])[None, :]


def _tc_prep(xp, w, asrc, adst):
    return pl.pallas_call(
        _prep_body,
        out_shape=[
            jax.ShapeDtypeStruct((NPAD, 16 + H1 * C1), jnp.float32),
            jax.ShapeDtypeStruct((NPAD, 16), jnp.float32),
            jax.ShapeDtypeStruct((1, 16), jnp.float32),
        ],
    )(xp, w, asrc, adst)


BR = NPAD // 8


def _mid_body(q_ref, b1_ref, w2_ref, asrc_ref, adst_ref,
              pk_ref, ad_ref, mh_ref, ms_ref, md_ref):
    @pl.when(pl.program_id(0) == 0)
    def _init():
        ms_ref[...] = jnp.full((1, 16), NEG, jnp.float32)
        md_ref[...] = jnp.full((1, 16), NEG, jnp.float32)

    den = q_ref[0, :, :16] + q_ref[1, :, :16]
    feat = q_ref[0, :, 16:] + q_ref[1, :, 16:]
    r = 1.0 / (den[:, :H1] + 1e-16)
    rrep = jnp.reshape(
        jnp.broadcast_to(r[:, :, None], (BR, H1, C1)), (BR, H1 * C1))
    o = feat * rrep + b1_ref[...]
    o = jnp.where(o > 0, o, jnp.exp(jnp.minimum(o, 0.0)) - 1.0)
    h2 = jnp.dot(o, w2_ref[...], precision=_HIGH)
    als = jnp.dot(h2, asrc_ref[...], precision=_HIGH)
    ald = jnp.dot(h2, adst_ref[...], precision=_HIGH)
    negpad = jnp.full((BR, 15), NEG, jnp.float32)
    h2pad = jnp.concatenate(
        [h2, jnp.zeros((BR, 16 - C2), jnp.float32)], axis=1)
    pk_ref[...] = jnp.concatenate([als, negpad, h2pad], axis=1)
    ad_ref[...] = jnp.concatenate([ald, negpad], axis=1)
    ms_ref[...] = jnp.maximum(ms_ref[...], jnp.max(als))
    md_ref[...] = jnp.maximum(md_ref[...], jnp.max(ald))
    mh_ref[...] = ms_ref[...] + md_ref[...]


def _tc_mid(q, b1, w2, asrc2, adst2):
    return pl.pallas_call(
        _mid_body,
        grid=(NPAD // BR,),
        in_specs=[
            pl.BlockSpec((2, BR, 16 + H1 * C1), lambda i: (0, i, 0)),
            pl.BlockSpec((1, H1 * C1), lambda i: (0, 0)),
            pl.BlockSpec((H1 * C1, C2), lambda i: (0, 0)),
            pl.BlockSpec((C2, H2), lambda i: (0, 0)),
            pl.BlockSpec((C2, H2), lambda i: (0, 0)),
        ],
        out_specs=[
            pl.BlockSpec((BR, 32), lambda i: (i, 0)),
            pl.BlockSpec((BR, 16), lambda i: (i, 0)),
            pl.BlockSpec((1, 16), lambda i: (0, 0)),
        ],
        out_shape=[
            jax.ShapeDtypeStruct((NPAD, 32), jnp.float32),
            jax.ShapeDtypeStruct((NPAD, 16), jnp.float32),
            jax.ShapeDtypeStruct((1, 16), jnp.float32),
        ],
        scratch_shapes=[
            pltpu.VMEM((1, 16), jnp.float32),
            pltpu.VMEM((1, 16), jnp.float32),
        ],
    )(q, b1, w2, asrc2, adst2)


def _final_body(q_ref, b2_ref, o_ref):
    den = q_ref[0, :N, 0:1] + q_ref[1, :N, 0:1]
    feat = q_ref[0, :N, 16:16 + C2] + q_ref[1, :N, 16:16 + C2]
    o_ref[...] = feat / (den + 1e-16) + b2_ref[...]


def _tc_final(q, b2):
    return pl.pallas_call(
        _final_body,
        out_shape=jax.ShapeDtypeStruct((N, C2), jnp.float32),
    )(q, b2.reshape(1, C2))


def _sc_fused(epad, npt, hv, chunk, unr):
    nch = npt // chunk
    aw = 16 * (hv + 1)
    mesh = plsc.VectorSubcoreMesh(core_axis_name="c", subcore_axis_name="s")

    @functools.partial(
        pl.kernel,
        out_type=jax.ShapeDtypeStruct((2, NPAD, aw), jnp.float32),
        mesh=mesh,
        compiler_params=pltpu.CompilerParams(use_tc_tiling_on_sc=False),
        scratch_types=[
            pltpu.VMEM((2, chunk), jnp.int32),
            pltpu.VMEM((2, chunk), jnp.int32),
            pltpu.VMEM((chunk, aw), jnp.float32),
            pltpu.VMEM((chunk, aw), jnp.float32),
            pltpu.VMEM((chunk, 16), jnp.float32),
            pltpu.VMEM((chunk, 16), jnp.float32),
            pltpu.VMEM((1, 16), jnp.float32),
            pltpu.VMEM_SHARED((NPAD, aw), jnp.float32),
            pltpu.SemaphoreType.DMA,
            pltpu.SemaphoreType.DMA,
        ],
    )
    def kern(pk_hbm, ad_hbm, mh_hbm, sd_hbm, z_hbm,
             out_hbm,
             idx0, idx1, pkr0, pkr1, adr0, adr1,
             mhv, acc, sem0, sem1):
        cid = lax.axis_index("c")
        sid = lax.axis_index("s")
        wid = cid * 16 + sid
        pltpu.sync_copy(z_hbm.at[pl.ds(sid * RPT, RPT)],
                        acc.at[pl.ds(sid * RPT, RPT)])
        pltpu.sync_copy(mh_hbm, mhv)
        plsc.subcore_barrier()
        mh = mhv[0, :]

        bufs = [(idx0, pkr0, adr0, sem0),
                (idx1, pkr1, adr1, sem1)]

        def fire(ci, s):
            idxv, pkr, adr, sem = bufs[s]
            row = (wid * nch + ci) * 2
            pltpu.sync_copy(sd_hbm.at[pl.ds(row, 2)], idxv)
            pltpu.async_copy(pk_hbm.at[idxv.at[0]], pkr, sem)
            pltpu.async_copy(ad_hbm.at[idxv.at[1]], adr, sem)

        def drain(s):
            idxv, pkr, adr, sem = bufs[s]
            pltpu.make_async_copy(pk_hbm.at[idxv.at[0]], pkr, sem).wait()
            pltpu.make_async_copy(ad_hbm.at[idxv.at[1]], adr, sem).wait()

        def compute(s):
            idxv, pkr, adr, sem = bufs[s]

            @plsc.parallel_loop(0, chunk, unroll=unr)
            def _edge(e):
                t = pkr[e, pl.ds(0, 16)] + adr[e, :]
                u = jnp.maximum(t, t * 0.2) - mh
                ex = jnp.exp(u)
                pkr[e, pl.ds(0, 16)] = ex
                for v in range(hv):
                    sp = jnp.broadcast_to(ex[v], (16,))
                    pkr[e, pl.ds(16 * (v + 1), 16)] = (
                        pkr[e, pl.ds(16 * (v + 1), 16)] * sp)

            pltpu.sync_copy(pkr, acc.at[idxv.at[1]], add=True)

        half = nch // 2
        fire(0, 0)

        @pl.loop(0, half)
        def _grp(g):
            fire(2 * g + 1, 1)
            drain(0)
            compute(0)
            fire(jnp.minimum(2 * g + 2, nch - 1), 0)
            drain(1)
            compute(1)

        drain(0)
        plsc.subcore_barrier()
        pltpu.sync_copy(acc.at[pl.ds(sid * RPT, RPT)],
                        out_hbm.at[cid, pl.ds(sid * RPT, RPT)])

    return kern


def _block_attn(a):
    h = a.shape[0]
    eye = jnp.eye(h, dtype=a.dtype)
    return (a[:, :, None] * eye[:, None, :]).reshape(h * a.shape[1], h)


def kernel(x, edge_index, W1, a_src1, a_dst1, b1, W2, a_src2, a_dst2, b2):
    e = edge_index.shape[1]
    etot = e + N
    npt = _cdiv(etot, NTILES * 5376) * 5376
    epad = npt * NTILES

    xp = jnp.concatenate([x, jnp.zeros((NPAD - N, D_IN), jnp.float32)], axis=0)
    loops = jnp.arange(N, dtype=jnp.int32)
    padv = N + jnp.arange(epad - etot, dtype=jnp.int32) % (NPAD - N)
    src = jnp.concatenate([edge_index[0], loops, padv])
    dst = jnp.concatenate([edge_index[1], loops, padv])

    asrc1 = _block_attn(a_src1)
    adst1 = _block_attn(a_dst1)
    asrc2 = _block_attn(a_src2)
    adst2 = _block_attn(a_dst2)

    z144 = jnp.zeros((NPAD, 16 + H1 * C1), jnp.float32)
    z32 = jnp.zeros((NPAD, 32), jnp.float32)

    def pack_idx(c):
        nch = npt // c
        s3 = src.reshape(NTILES, nch, c)
        d3 = dst.reshape(NTILES, nch, c)
        return jnp.stack([s3, d3], axis=2).reshape(NTILES * nch * 2, c)

    sd1 = pack_idx(112)
    sd2 = pack_idx(384)

    fused1 = _sc_fused(epad, npt, H1, 112, 4)
    fused2 = _sc_fused(epad, npt, 1, 384, 8)

    pk1, ad1, mh1 = _tc_prep(xp, W1, asrc1, adst1)
    q1 = fused1(pk1, ad1, mh1, sd1, z144)

    pk2, ad2, mh2 = _tc_mid(q1, b1.reshape(1, H1 * C1), W2, asrc2, adst2)
    q2 = fused2(pk2, ad2, mh2, sd2, z32)

    return _tc_final(q2, b2)
